# baseline (speedup 1.0000x reference)
import functools

import jax
import jax.numpy as jnp
from jax import lax
from jax.experimental import pallas as pl
from jax.experimental.pallas import tpu as pltpu
from jax.experimental.pallas import tpu_sc as plsc

_VOCAB = 128
_B, _S, _W = 64, 256, 16
_NC, _NS = 2, 16
_NW = _NC * _NS
_BPW = _B // _NW


def _fofe_body(ids_hbm, pow_hbm, out_hbm, ids_v, pow_v, out_v, sem):
    wid = lax.axis_index("s") * _NC + lax.axis_index("c")
    base = wid * _BPW

    pltpu.sync_copy(ids_hbm.at[pl.ds(base, _BPW)], ids_v)
    pltpu.sync_copy(pow_hbm, pow_v)
    pvec = pow_v[...]
    zero16 = jnp.zeros((16,), jnp.float32)

    copies = []
    for sent in range(_BPW):
        sentv = jnp.full((16,), sent, jnp.int32)

        def row_body(s, carry, sent=sent, sentv=sentv):
            for c in range(_VOCAB // 16):
                out_v[sent, s, pl.ds(c * 16, 16)] = zero16
            idx = ids_v[sent, s, :]
            sv = jnp.full((16,), s, jnp.int32)
            plsc.addupdate_scatter(out_v, [sentv, sv, idx], pvec)
            return carry

        lax.fori_loop(0, _S, row_body, 0, unroll=8)
        cp = pltpu.make_async_copy(
            out_v.at[sent], out_hbm.at[base + sent], sem
        )
        cp.start()
        copies.append(cp)
    for cp in copies:
        cp.wait()


@jax.jit
def _fofe(ids, powers):
    mesh = plsc.VectorSubcoreMesh(core_axis_name="c", subcore_axis_name="s")
    run = functools.partial(
        pl.kernel,
        mesh=mesh,
        out_type=jax.ShapeDtypeStruct((_B, _S, _VOCAB), jnp.float32),
        scratch_types=[
            pltpu.VMEM((_BPW, _S, _W), jnp.int32),
            pltpu.VMEM((_W,), jnp.float32),
            pltpu.VMEM((_BPW, _S, _VOCAB), jnp.float32),
            pltpu.SemaphoreType.DMA,
        ],
        compiler_params=pltpu.CompilerParams(
            needs_layout_passes=False, use_tc_tiling_on_sc=False
        ),
    )(_fofe_body)
    return run(ids, powers)


def kernel(sents, lengths, forgetting_factor):
    f = forgetting_factor[0]
    powers = f ** jnp.arange(_W - 1, -1, -1, dtype=jnp.float32)
    out = _fofe(sents, powers)
    return (out, lengths)
